# Initial kernel scaffold; baseline (speedup 1.0000x reference)
import jax
import jax.numpy as jnp
from jax.experimental import pallas as pl


def kernel(x, ctrs, Wv, Ov):
    raise NotImplementedError("write your pallas kernel here")


# speedup vs baseline: 1.7880x; 1.7880x over previous
import functools

import jax
import jax.numpy as jnp
from jax.experimental import pallas as pl

N_TOKENS = 8192
C = 64
K = 8
D_IN = 64
D_OUT = 64


def _fused_body(x_ref, ctrs_ref, wvt_ref, ov_ref, out_ref):
    x = x_ref[...]
    ctrs = ctrs_ref[...]
    T = x.shape[0]

    xc = jax.lax.dot_general(
        x, ctrs, dimension_numbers=(((1,), (1,)), ((), ())),
        precision=jax.lax.Precision.HIGHEST,
        preferred_element_type=jnp.float32)
    x_sq = jnp.sum(x * x, axis=1, keepdims=True)
    c_sq = jnp.sum(ctrs * ctrs, axis=1)[None, :]
    d = x_sq - 2.0 * xc + c_sq

    colid = jax.lax.broadcasted_iota(jnp.int32, (T, C), 1)
    m0 = jnp.min(d, axis=1, keepdims=True)
    work = d
    mask = jnp.zeros((T, C), jnp.bool_)
    for _ in range(K):
        mk = jnp.min(work, axis=1, keepdims=True)
        is_min = work == mk
        sel_idx = jnp.min(jnp.where(is_min, colid, C), axis=1, keepdims=True)
        sel = colid == sel_idx
        mask = jnp.logical_or(mask, sel)
        work = jnp.where(sel, jnp.float32(jnp.inf), work)

    e = jnp.where(mask, jnp.exp(-(d - m0)), 0.0)
    a = e / jnp.sum(e, axis=1, keepdims=True)

    y = jax.lax.dot_general(
        x, wvt_ref[...], dimension_numbers=(((1,), (0,)), ((), ())),
        precision=jax.lax.Precision.HIGHEST,
        preferred_element_type=jnp.float32)
    acc = jax.lax.dot_general(
        a, ov_ref[...], dimension_numbers=(((1,), (0,)), ((), ())),
        preferred_element_type=jnp.float32)
    for c in range(C):
        acc = acc + a[:, c:c + 1] * y[:, c * D_OUT:(c + 1) * D_OUT]
    out_ref[...] = acc


@jax.jit
def kernel(x, ctrs, Wv, Ov):
    n = x.shape[0]
    tile = 256
    grid = (n // tile,)
    wvt = jnp.transpose(Wv, (1, 0, 2)).reshape(D_IN, C * D_OUT)
    return pl.pallas_call(
        _fused_body,
        grid=grid,
        in_specs=[
            pl.BlockSpec((tile, D_IN), lambda i: (i, 0)),
            pl.BlockSpec((C, D_IN), lambda i: (0, 0)),
            pl.BlockSpec((D_IN, C * D_OUT), lambda i: (0, 0)),
            pl.BlockSpec((C, D_OUT), lambda i: (0, 0)),
        ],
        out_specs=pl.BlockSpec((tile, D_OUT), lambda i: (i, 0)),
        out_shape=jax.ShapeDtypeStruct((n, D_OUT), jnp.float32),
    )(x, ctrs, wvt, Ov)
